# Initial kernel scaffold; baseline (speedup 1.0000x reference)
import jax
import jax.numpy as jnp
from jax.experimental import pallas as pl


def kernel(x, edge_index, W1, b1, W2, b2):
    raise NotImplementedError("write your pallas kernel here")


# speedup vs baseline: 142.0732x; 142.0732x over previous
import functools

import jax
import jax.numpy as jnp
from jax import lax
from jax.experimental import pallas as pl
from jax.experimental.pallas import tpu as pltpu
from jax.experimental.pallas import tpu_sc as plsc

_LANE = 128
_PAD_SPREAD = 2048
_K = 8


def _make_deg_pass(n_pad, rows, nc, ns):
    nw = nc * ns
    cpt = rows // (nw * _K)
    seg = n_pad // ns
    mesh = plsc.VectorSubcoreMesh(core_axis_name="c", subcore_axis_name="s")

    @functools.partial(
        pl.kernel,
        mesh=mesh,
        out_type=jax.ShapeDtypeStruct((nc, n_pad), jnp.float32),
        scratch_types=[
            pltpu.VMEM((_K, _LANE), jnp.int32),
            pltpu.VMEM((_K, _LANE), jnp.float32),
            pltpu.VMEM_SHARED((n_pad,), jnp.float32),
        ],
    )
    def k(dst_hbm, ones_hbm, zeros_hbm, out_hbm, dst_v, val_v, acc_sh):
        cid = lax.axis_index("c")
        sid = lax.axis_index("s")
        pltpu.sync_copy(zeros_hbm.at[pl.ds(sid * seg, seg)],
                        acc_sh.at[pl.ds(sid * seg, seg)])
        pltpu.sync_copy(ones_hbm, val_v)
        plsc.subcore_barrier()
        row0 = (cid * ns + sid) * (cpt * _K)

        def body(i, carry):
            r = row0 + i * _K
            pltpu.sync_copy(dst_hbm.at[pl.ds(r, _K)], dst_v)
            for j in range(_K):
                pltpu.sync_copy(val_v.at[j], acc_sh.at[dst_v.at[j]], add=True)
            return carry

        lax.fori_loop(0, cpt, body, 0)
        plsc.subcore_barrier()
        pltpu.sync_copy(acc_sh.at[pl.ds(sid * seg, seg)],
                        out_hbm.at[cid, pl.ds(sid * seg, seg)])

    return k


def _make_gs_pass(n_pad, rows, nc, ns):
    nw = nc * ns
    cpt = rows // (nw * _K)
    seg = n_pad // ns
    mesh = plsc.VectorSubcoreMesh(core_axis_name="c", subcore_axis_name="s")

    @functools.partial(
        pl.kernel,
        mesh=mesh,
        out_type=jax.ShapeDtypeStruct((nc, n_pad), jnp.float32),
        scratch_types=[
            pltpu.VMEM((_K, _LANE), jnp.int32),
            pltpu.VMEM((_K, _LANE), jnp.int32),
            pltpu.VMEM((_K, _LANE), jnp.float32),
            pltpu.VMEM_SHARED((n_pad,), jnp.float32),
            pltpu.VMEM_SHARED((n_pad,), jnp.float32),
            pltpu.SemaphoreType.DMA,
        ],
    )
    def k(src_hbm, dst_hbm, table_hbm, zeros_hbm, out_hbm,
          src_v, dst_v, val_v, table_sh, acc_sh, sem):
        cid = lax.axis_index("c")
        sid = lax.axis_index("s")
        pltpu.sync_copy(table_hbm.at[pl.ds(sid * seg, seg)],
                        table_sh.at[pl.ds(sid * seg, seg)])
        pltpu.sync_copy(zeros_hbm.at[pl.ds(sid * seg, seg)],
                        acc_sh.at[pl.ds(sid * seg, seg)])
        plsc.subcore_barrier()
        row0 = (cid * ns + sid) * (cpt * _K)

        def body(i, carry):
            r = row0 + i * _K
            pltpu.sync_copy(src_hbm.at[pl.ds(r, _K)], src_v)
            pltpu.sync_copy(dst_hbm.at[pl.ds(r, _K)], dst_v)
            copies = [
                pltpu.async_copy(table_sh.at[src_v.at[j]], val_v.at[j], sem)
                for j in range(_K)
            ]
            for c in copies:
                c.wait()
            for j in range(_K):
                pltpu.sync_copy(val_v.at[j], acc_sh.at[dst_v.at[j]], add=True)
            return carry

        lax.fori_loop(0, cpt, body, 0)
        plsc.subcore_barrier()
        pltpu.sync_copy(acc_sh.at[pl.ds(sid * seg, seg)],
                        out_hbm.at[cid, pl.ds(sid * seg, seg)])

    return k


def _node_pass1(degp, x2d):
    nc, r, l = degp.shape

    def body(degp_ref, x_ref, dinv_ref, dinvx_ref):
        deg = degp_ref[0]
        for c in range(1, nc):
            deg = deg + degp_ref[c]
        deg = deg + 1.0
        dinv = lax.rsqrt(deg)
        dinv_ref[...] = dinv
        dinvx_ref[...] = dinv * x_ref[...]

    return pl.pallas_call(
        body,
        out_shape=[jax.ShapeDtypeStruct((r, l), jnp.float32),
                   jax.ShapeDtypeStruct((r, l), jnp.float32)],
    )(degp, x2d)


def _node_pass2(accp, dinv2d, x2d, W1, b1, W2):
    nc, r, l = accp.shape
    f = W1.shape[1]

    def body(accp_ref, dinv_ref, x_ref, w1_ref, b1_ref, w2_ref, t_ref, dinvt_ref):
        acc = accp_ref[0]
        for c in range(1, nc):
            acc = acc + accp_ref[c]
        dinv = dinv_ref[...]
        s1 = dinv * (acc + dinv * x_ref[...])
        t = jnp.zeros((r, l), jnp.float32)
        for k in range(f):
            t = t + jnp.maximum(s1 * w1_ref[0, k] + b1_ref[k], 0.0) * w2_ref[k, 0]
        t_ref[...] = t
        dinvt_ref[...] = dinv * t

    return pl.pallas_call(
        body,
        in_specs=[pl.BlockSpec(memory_space=pltpu.VMEM)] * 3
        + [pl.BlockSpec(memory_space=pltpu.SMEM)] * 3,
        out_shape=[jax.ShapeDtypeStruct((r, l), jnp.float32),
                   jax.ShapeDtypeStruct((r, l), jnp.float32)],
    )(accp, dinv2d, x2d, W1, b1, W2)


def _node_pass3(acc2p, dinv2d, t2d, b2):
    nc, r, l = acc2p.shape

    def body(accp_ref, dinv_ref, t_ref, b2_ref, out_ref):
        acc = accp_ref[0]
        for c in range(1, nc):
            acc = acc + accp_ref[c]
        dinv = dinv_ref[...]
        out_ref[...] = dinv * (acc + dinv * t_ref[...]) + b2_ref[0]

    return pl.pallas_call(
        body,
        in_specs=[pl.BlockSpec(memory_space=pltpu.VMEM)] * 3
        + [pl.BlockSpec(memory_space=pltpu.SMEM)],
        out_shape=jax.ShapeDtypeStruct((r, l), jnp.float32),
    )(acc2p, dinv2d, t2d, b2)


def kernel(x, edge_index, W1, b1, W2, b2):
    n = x.shape[0]
    e = edge_index.shape[1]
    info = plsc.get_sparse_core_info()
    nc, ns = info.num_cores, info.num_subcores
    nw = nc * ns

    n_pad = ((n + _PAD_SPREAD + _LANE - 1) // _LANE) * _LANE
    n_pad = ((n_pad + _LANE * ns - 1) // (_LANE * ns)) * (_LANE * ns)
    nr = n_pad // _LANE

    chunk_e = _K * _LANE
    cpt = -(-e // (chunk_e * nw))
    rows = cpt * nw * _K
    e_pad = rows * _LANE
    pad = e_pad - e

    src = edge_index[0]
    dst = edge_index[1]
    pad_idx = (n + (jnp.arange(pad, dtype=jnp.int32) % _PAD_SPREAD)).astype(jnp.int32)
    src2d = jnp.concatenate([src, pad_idx]).reshape(rows, _LANE)
    dst2d = jnp.concatenate([dst, pad_idx]).reshape(rows, _LANE)

    xf = jnp.concatenate([x[:, 0], jnp.zeros((n_pad - n,), jnp.float32)])
    x2d = xf.reshape(nr, _LANE)
    zeros = jnp.zeros((n_pad,), jnp.float32)
    ones = jnp.ones((_K, _LANE), jnp.float32)

    deg_pass = _make_deg_pass(n_pad, rows, nc, ns)
    gs_pass = _make_gs_pass(n_pad, rows, nc, ns)

    degp = deg_pass(dst2d, ones, zeros).reshape(nc, nr, _LANE)
    dinv2d, dinvx2d = _node_pass1(degp, x2d)

    accp = gs_pass(src2d, dst2d, dinvx2d.reshape(n_pad), zeros)
    t2d, dinvt2d = _node_pass2(accp.reshape(nc, nr, _LANE), dinv2d, x2d, W1, b1, W2)

    acc2p = gs_pass(src2d, dst2d, dinvt2d.reshape(n_pad), zeros)
    out2d = _node_pass3(acc2p.reshape(nc, nr, _LANE), dinv2d, t2d, b2)

    return out2d.reshape(n_pad)[:n].reshape(n, 1)


# speedup vs baseline: 273.4127x; 1.9245x over previous
import functools

import jax
import jax.numpy as jnp
from jax import lax
from jax.experimental import pallas as pl
from jax.experimental.pallas import tpu as pltpu
from jax.experimental.pallas import tpu_sc as plsc

_LANE = 128
_PAD_SPREAD = 2048
_KG = 8
_KD = 32

_SC_PARAMS = pltpu.CompilerParams(needs_layout_passes=False)


def _zero_acc(acc_v, n_pad):
    z = jnp.zeros((16,), jnp.float32)

    def zbody(i, c):
        for u in range(8):
            acc_v[pl.ds((i * 8 + u) * 16, 16)] = z
        return c

    lax.fori_loop(0, n_pad // 128, zbody, 0)


def _local_scatter(acc_v, dst_v, val_v, b, k):
    for j in range(k):
        for u in range(8):
            idx = dst_v[b, j, pl.ds(u * 16, 16)]
            val = val_v[b, j, pl.ds(u * 16, 16)]
            plsc.addupdate_scatter(acc_v, [idx], val)


def _make_deg_pass(n_pad, rows, nc, ns):
    nw = nc * ns
    cpt = rows // (nw * _KD)
    assert cpt % 2 == 0
    mesh = plsc.VectorSubcoreMesh(core_axis_name="c", subcore_axis_name="s")

    @functools.partial(
        pl.kernel,
        mesh=mesh,
        out_type=jax.ShapeDtypeStruct((nw, n_pad), jnp.float32),
        scratch_types=[
            pltpu.VMEM((2, _KD, _LANE), jnp.int32),
            pltpu.VMEM((n_pad,), jnp.float32),
            pltpu.SemaphoreType.DMA,
            pltpu.SemaphoreType.DMA,
        ],
        compiler_params=_SC_PARAMS,
    )
    def k(dst_hbm, out_hbm, dst_v, acc_v, sem0, sem1):
        cid = lax.axis_index("c")
        sid = lax.axis_index("s")
        wid = cid * ns + sid
        row0 = wid * (cpt * _KD)
        ones = jnp.ones((16,), jnp.float32)
        sems = (sem0, sem1)

        _zero_acc(acc_v, n_pad)
        pltpu.async_copy(dst_hbm.at[pl.ds(row0, _KD)], dst_v.at[0], sem0)

        def scatter_ones(b):
            for j in range(_KD):
                for u in range(8):
                    idx = dst_v[b, j, pl.ds(u * 16, 16)]
                    plsc.addupdate_scatter(acc_v, [idx], ones)

        def body(p, carry):
            r = row0 + (2 * p) * _KD
            pltpu.async_copy(dst_hbm.at[pl.ds(r + _KD, _KD)], dst_v.at[1], sem1)
            pltpu.make_async_copy(dst_hbm.at[pl.ds(r, _KD)], dst_v.at[0], sem0).wait()
            scatter_ones(0)

            @pl.when(p < cpt // 2 - 1)
            def _():
                pltpu.async_copy(dst_hbm.at[pl.ds(r + 2 * _KD, _KD)], dst_v.at[0], sem0)

            pltpu.make_async_copy(dst_hbm.at[pl.ds(r + _KD, _KD)], dst_v.at[1], sem1).wait()
            scatter_ones(1)
            return carry

        lax.fori_loop(0, cpt // 2, body, 0)
        pltpu.sync_copy(acc_v, out_hbm.at[wid])

    return k


def _make_gs_pass(n_pad, rows, nc, ns):
    nw = nc * ns
    cpt = rows // (nw * _KG)
    assert cpt % 2 == 0
    seg = n_pad // ns
    mesh = plsc.VectorSubcoreMesh(core_axis_name="c", subcore_axis_name="s")

    @functools.partial(
        pl.kernel,
        mesh=mesh,
        out_type=jax.ShapeDtypeStruct((nw, n_pad), jnp.float32),
        scratch_types=[
            pltpu.VMEM((2, _KG, _LANE), jnp.int32),
            pltpu.VMEM((2, _KG, _LANE), jnp.int32),
            pltpu.VMEM((2, _KG, _LANE), jnp.float32),
            pltpu.VMEM((n_pad,), jnp.float32),
            pltpu.VMEM_SHARED((n_pad,), jnp.float32),
            pltpu.SemaphoreType.DMA,
            pltpu.SemaphoreType.DMA,
            pltpu.SemaphoreType.DMA,
            pltpu.SemaphoreType.DMA,
        ],
        compiler_params=_SC_PARAMS,
    )
    def k(src_hbm, dst_hbm, table_hbm, out_hbm,
          src_v, dst_v, val_v, acc_v, table_sh, gsem0, gsem1, isem0, isem1):
        cid = lax.axis_index("c")
        sid = lax.axis_index("s")
        wid = cid * ns + sid
        row0 = wid * (cpt * _KG)
        isems = (isem0, isem1)
        gsems = (gsem0, gsem1)

        pltpu.sync_copy(table_hbm.at[pl.ds(sid * seg, seg)],
                        table_sh.at[pl.ds(sid * seg, seg)])
        _zero_acc(acc_v, n_pad)
        plsc.subcore_barrier()

        def load_idx(chunk, b):
            r = row0 + chunk * _KG
            pltpu.async_copy(src_hbm.at[pl.ds(r, _KG)], src_v.at[b], isems[b])
            pltpu.async_copy(dst_hbm.at[pl.ds(r, _KG)], dst_v.at[b], isems[b])

        def wait_idx(chunk, b):
            r = row0 + chunk * _KG
            pltpu.make_async_copy(src_hbm.at[pl.ds(r, _KG)], src_v.at[b], isems[b]).wait()
            pltpu.make_async_copy(dst_hbm.at[pl.ds(r, _KG)], dst_v.at[b], isems[b]).wait()

        def fire_gathers(b):
            for j in range(_KG):
                pltpu.async_copy(table_sh.at[src_v.at[b, j]], val_v.at[b, j], gsems[b])

        def drain_gathers(b):
            for j in range(_KG):
                pltpu.make_async_copy(table_sh.at[src_v.at[b, j]],
                                      val_v.at[b, j], gsems[b]).wait()

        load_idx(0, 0)
        wait_idx(0, 0)
        fire_gathers(0)
        load_idx(1, 1)

        def body(p, carry):
            c0 = 2 * p

            wait_idx(c0 + 1, 1)
            fire_gathers(1)
            drain_gathers(0)
            _local_scatter(acc_v, dst_v, val_v, 0, _KG)

            @pl.when(p < cpt // 2 - 1)
            def _():
                load_idx(c0 + 2, 0)
                wait_idx(c0 + 2, 0)
                fire_gathers(0)

            drain_gathers(1)
            _local_scatter(acc_v, dst_v, val_v, 1, _KG)

            @pl.when(p < cpt // 2 - 1)
            def _():
                load_idx(c0 + 3, 1)
            return carry

        lax.fori_loop(0, cpt // 2, body, 0)
        pltpu.sync_copy(acc_v, out_hbm.at[wid])

    return k


def _node_pass1(degp, x2d):
    nw, r, l = degp.shape

    def body(degp_ref, x_ref, dinv_ref, dinvx_ref):
        deg = degp_ref[0]
        for c in range(1, nw):
            deg = deg + degp_ref[c]
        deg = deg + 1.0
        dinv = lax.rsqrt(deg)
        dinv_ref[...] = dinv
        dinvx_ref[...] = dinv * x_ref[...]

    return pl.pallas_call(
        body,
        out_shape=[jax.ShapeDtypeStruct((r, l), jnp.float32),
                   jax.ShapeDtypeStruct((r, l), jnp.float32)],
    )(degp, x2d)


def _node_pass2(accp, dinv2d, x2d, W1, b1, W2):
    nw, r, l = accp.shape
    f = W1.shape[1]

    def body(accp_ref, dinv_ref, x_ref, w1_ref, b1_ref, w2_ref, t_ref, dinvt_ref):
        acc = accp_ref[0]
        for c in range(1, nw):
            acc = acc + accp_ref[c]
        dinv = dinv_ref[...]
        s1 = dinv * (acc + dinv * x_ref[...])
        t = jnp.zeros((r, l), jnp.float32)
        for k in range(f):
            t = t + jnp.maximum(s1 * w1_ref[0, k] + b1_ref[k], 0.0) * w2_ref[k, 0]
        t_ref[...] = t
        dinvt_ref[...] = dinv * t

    return pl.pallas_call(
        body,
        in_specs=[pl.BlockSpec(memory_space=pltpu.VMEM)] * 3
        + [pl.BlockSpec(memory_space=pltpu.SMEM)] * 3,
        out_shape=[jax.ShapeDtypeStruct((r, l), jnp.float32),
                   jax.ShapeDtypeStruct((r, l), jnp.float32)],
    )(accp, dinv2d, x2d, W1, b1, W2)


def _node_pass3(acc2p, dinv2d, t2d, b2):
    nw, r, l = acc2p.shape

    def body(accp_ref, dinv_ref, t_ref, b2_ref, out_ref):
        acc = accp_ref[0]
        for c in range(1, nw):
            acc = acc + accp_ref[c]
        dinv = dinv_ref[...]
        out_ref[...] = dinv * (acc + dinv * t_ref[...]) + b2_ref[0]

    return pl.pallas_call(
        body,
        in_specs=[pl.BlockSpec(memory_space=pltpu.VMEM)] * 3
        + [pl.BlockSpec(memory_space=pltpu.SMEM)],
        out_shape=jax.ShapeDtypeStruct((r, l), jnp.float32),
    )(acc2p, dinv2d, t2d, b2)


def kernel(x, edge_index, W1, b1, W2, b2):
    n = x.shape[0]
    e = edge_index.shape[1]
    info = plsc.get_sparse_core_info()
    nc, ns = info.num_cores, info.num_subcores
    nw = nc * ns

    n_pad = ((n + _PAD_SPREAD + _LANE - 1) // _LANE) * _LANE
    n_pad = ((n_pad + _LANE * ns - 1) // (_LANE * ns)) * (_LANE * ns)
    nr = n_pad // _LANE

    unit = max(2 * _KG, 2 * _KD) * nw
    rows = -(-e // (_LANE * unit)) * unit
    e_pad = rows * _LANE
    pad = e_pad - e

    src = edge_index[0]
    dst = edge_index[1]
    pad_idx = (n + (jnp.arange(pad, dtype=jnp.int32) % _PAD_SPREAD)).astype(jnp.int32)
    src2d = jnp.concatenate([src, pad_idx]).reshape(rows, _LANE)
    dst2d = jnp.concatenate([dst, pad_idx]).reshape(rows, _LANE)

    xf = jnp.concatenate([x[:, 0], jnp.zeros((n_pad - n,), jnp.float32)])
    x2d = xf.reshape(nr, _LANE)

    deg_pass = _make_deg_pass(n_pad, rows, nc, ns)
    gs_pass = _make_gs_pass(n_pad, rows, nc, ns)

    degp = deg_pass(dst2d).reshape(nw, nr, _LANE)
    dinv2d, dinvx2d = _node_pass1(degp, x2d)

    accp = gs_pass(src2d, dst2d, dinvx2d.reshape(n_pad))
    t2d, dinvt2d = _node_pass2(accp.reshape(nw, nr, _LANE), dinv2d, x2d, W1, b1, W2)

    acc2p = gs_pass(src2d, dst2d, dinvt2d.reshape(n_pad))
    out2d = _node_pass3(acc2p.reshape(nw, nr, _LANE), dinv2d, t2d, b2)

    return out2d.reshape(n_pad)[:n].reshape(n, 1)


# speedup vs baseline: 420.6902x; 1.5387x over previous
import functools

import jax
import jax.numpy as jnp
from jax import lax
from jax.experimental import pallas as pl
from jax.experimental.pallas import tpu as pltpu
from jax.experimental.pallas import tpu_sc as plsc

_LANE = 128
_K = 8
_NBUF = 4

_SC_PARAMS = pltpu.CompilerParams(needs_layout_passes=False)


def _zero_acc(acc_v, n_pad):
    z = jnp.zeros((16,), jnp.float32)

    def zbody(i, c):
        for u in range(8):
            acc_v[pl.ds((i * 8 + u) * 16, 16)] = z
        return c

    lax.fori_loop(0, n_pad // 128, zbody, 0)


def _make_edge_pass(n_pad, real_rows, rows, nc, ns, with_gather):
    nw = nc * ns
    cpt = rows // (nw * _K)
    assert cpt % _NBUF == 0 and cpt > _NBUF
    seg = n_pad // ns
    mesh = plsc.VectorSubcoreMesh(core_axis_name="c", subcore_axis_name="s")

    scratch = [
        pltpu.VMEM((_NBUF, _K, _LANE), jnp.int32),
        pltpu.VMEM((n_pad,), jnp.float32),
    ] + [pltpu.SemaphoreType.DMA] * _NBUF
    if with_gather:
        scratch += [
            pltpu.VMEM((_NBUF, _K, _LANE), jnp.int32),
            pltpu.VMEM((_NBUF, _K, _LANE), jnp.float32),
            pltpu.VMEM_SHARED((n_pad,), jnp.float32),
        ] + [pltpu.SemaphoreType.DMA] * _NBUF

    @functools.partial(
        pl.kernel,
        mesh=mesh,
        out_type=jax.ShapeDtypeStruct((nw, n_pad), jnp.float32),
        scratch_types=scratch,
        compiler_params=_SC_PARAMS,
    )
    def k(*args):
        if with_gather:
            (eidx_hbm, pad_hbm, table_hbm, out_hbm, dst_v, acc_v,
             *rest) = args
            isems = rest[:_NBUF]
            src_v, val_v = rest[_NBUF], rest[_NBUF + 1]
            table_sh = rest[_NBUF + 2]
            gsems = rest[_NBUF + 3:]
        else:
            eidx_hbm, pad_hbm, out_hbm, dst_v, acc_v, *isems = args

        cid = lax.axis_index("c")
        sid = lax.axis_index("s")
        wid = cid * ns + sid
        c_base = wid * cpt
        ones = jnp.ones((16,), jnp.float32)

        if with_gather:
            pltpu.sync_copy(table_hbm.at[pl.ds(sid * seg, seg)],
                            table_sh.at[pl.ds(sid * seg, seg)])
        _zero_acc(acc_v, n_pad)
        if with_gather:
            plsc.subcore_barrier()

        def load_idx(c, b):
            r = c * _K

            @pl.when(r < real_rows)
            def _():
                pltpu.async_copy(eidx_hbm.at[1, pl.ds(r, _K)], dst_v.at[b],
                                 isems[b])
                if with_gather:
                    pltpu.async_copy(eidx_hbm.at[0, pl.ds(r, _K)],
                                     src_v.at[b], isems[b])

            @pl.when(r >= real_rows)
            def _():
                pltpu.async_copy(pad_hbm.at[pl.ds(r - real_rows, _K)],
                                 dst_v.at[b], isems[b])
                if with_gather:
                    pltpu.async_copy(pad_hbm.at[pl.ds(r - real_rows, _K)],
                                     src_v.at[b], isems[b])

        def wait_idx(b):
            pltpu.make_async_copy(eidx_hbm.at[1, pl.ds(0, _K)], dst_v.at[b],
                                  isems[b]).wait()
            if with_gather:
                pltpu.make_async_copy(eidx_hbm.at[0, pl.ds(0, _K)],
                                      src_v.at[b], isems[b]).wait()

        def fire_gathers(b):
            for j in range(_K):
                pltpu.async_copy(table_sh.at[src_v.at[b, j]], val_v.at[b, j],
                                 gsems[b])

        def drain_gathers(b):
            for j in range(_K):
                pltpu.make_async_copy(table_sh.at[src_v.at[b, j]],
                                      val_v.at[b, j], gsems[b]).wait()

        def consume(b):
            for j in range(_K):
                for u in range(8):
                    idx = dst_v[b, j, pl.ds(u * 16, 16)]
                    if with_gather:
                        val = val_v[b, j, pl.ds(u * 16, 16)]
                    else:
                        val = ones
                    plsc.addupdate_scatter(acc_v, [idx], val)

        for b in range(_NBUF):
            load_idx(c_base + b, b)

        def body(q, carry):
            c0 = c_base + q * _NBUF
            if with_gather:
                for b in range(_NBUF):
                    wait_idx(b)
                    fire_gathers(b)
                for b in range(_NBUF):
                    drain_gathers(b)
                    consume(b)

                    @pl.when(q < cpt // _NBUF - 1)
                    def _(b=b):
                        load_idx(c0 + b + _NBUF, b)
            else:
                for b in range(_NBUF):
                    wait_idx(b)
                    consume(b)

                    @pl.when(q < cpt // _NBUF - 1)
                    def _(b=b):
                        load_idx(c0 + b + _NBUF, b)
            return carry

        lax.fori_loop(0, cpt // _NBUF, body, 0)
        pltpu.sync_copy(acc_v, out_hbm.at[wid])

    return k


def _node_pass1(degp, x2d):
    nw, r, l = degp.shape

    def body(degp_ref, x_ref, dinv_ref, dinvx_ref):
        deg = degp_ref[0]
        for c in range(1, nw):
            deg = deg + degp_ref[c]
        deg = deg + 1.0
        dinv = lax.rsqrt(deg)
        dinv_ref[...] = dinv
        dinvx_ref[...] = dinv * x_ref[...]

    return pl.pallas_call(
        body,
        out_shape=[jax.ShapeDtypeStruct((r, l), jnp.float32),
                   jax.ShapeDtypeStruct((r, l), jnp.float32)],
    )(degp, x2d)


def _node_pass2(accp, dinv2d, x2d, W1, b1, W2):
    nw, r, l = accp.shape
    f = W1.shape[1]

    def body(accp_ref, dinv_ref, x_ref, w1_ref, b1_ref, w2_ref, t_ref, dinvt_ref):
        acc = accp_ref[0]
        for c in range(1, nw):
            acc = acc + accp_ref[c]
        dinv = dinv_ref[...]
        s1 = dinv * (acc + dinv * x_ref[...])
        t = jnp.zeros((r, l), jnp.float32)
        for k in range(f):
            t = t + jnp.maximum(s1 * w1_ref[0, k] + b1_ref[k], 0.0) * w2_ref[k, 0]
        t_ref[...] = t
        dinvt_ref[...] = dinv * t

    return pl.pallas_call(
        body,
        in_specs=[pl.BlockSpec(memory_space=pltpu.VMEM)] * 3
        + [pl.BlockSpec(memory_space=pltpu.SMEM)] * 3,
        out_shape=[jax.ShapeDtypeStruct((r, l), jnp.float32),
                   jax.ShapeDtypeStruct((r, l), jnp.float32)],
    )(accp, dinv2d, x2d, W1, b1, W2)


def _node_pass3(acc2p, dinv2d, t2d, b2):
    nw, r, l = acc2p.shape

    def body(accp_ref, dinv_ref, t_ref, b2_ref, out_ref):
        acc = accp_ref[0]
        for c in range(1, nw):
            acc = acc + accp_ref[c]
        dinv = dinv_ref[...]
        out_ref[...] = dinv * (acc + dinv * t_ref[...]) + b2_ref[0]

    return pl.pallas_call(
        body,
        in_specs=[pl.BlockSpec(memory_space=pltpu.VMEM)] * 3
        + [pl.BlockSpec(memory_space=pltpu.SMEM)],
        out_shape=jax.ShapeDtypeStruct((r, l), jnp.float32),
    )(acc2p, dinv2d, t2d, b2)


def kernel(x, edge_index, W1, b1, W2, b2):
    n = x.shape[0]
    e = edge_index.shape[1]
    assert e % _LANE == 0
    info = plsc.get_sparse_core_info()
    nc, ns = info.num_cores, info.num_subcores
    nw = nc * ns

    n_pad = ((n + 256 + _LANE * ns - 1) // (_LANE * ns)) * (_LANE * ns)
    spread = n_pad - n
    nr = n_pad // _LANE

    real_rows = e // _LANE
    unit = _NBUF * _K * nw
    rows = -(-real_rows // unit) * unit
    pad_rows = rows - real_rows

    eidx3d = edge_index.reshape(2, real_rows, _LANE)
    pad2d = (n + (jnp.arange(pad_rows * _LANE, dtype=jnp.int32) % spread)
             ).reshape(pad_rows, _LANE)

    xf = jnp.concatenate([x[:, 0], jnp.zeros((n_pad - n,), jnp.float32)])
    x2d = xf.reshape(nr, _LANE)

    deg_pass = _make_edge_pass(n_pad, real_rows, rows, nc, ns, False)
    gs_pass = _make_edge_pass(n_pad, real_rows, rows, nc, ns, True)

    degp = deg_pass(eidx3d, pad2d).reshape(nw, nr, _LANE)
    dinv2d, dinvx2d = _node_pass1(degp, x2d)

    accp = gs_pass(eidx3d, pad2d, dinvx2d.reshape(n_pad))
    t2d, dinvt2d = _node_pass2(accp.reshape(nw, nr, _LANE), dinv2d, x2d, W1, b1, W2)

    acc2p = gs_pass(eidx3d, pad2d, dinvt2d.reshape(n_pad))
    out2d = _node_pass3(acc2p.reshape(nw, nr, _LANE), dinv2d, t2d, b2)

    return out2d.reshape(n_pad)[:n].reshape(n, 1)
